# baseline (speedup 1.0000x reference)
import numpy as np
import jax
import jax.numpy as jnp
from jax.experimental import pallas as pl
from jax.experimental.pallas import tpu as pltpu

THRESH = 0.5
DECAY = 0.2
TW = 20

_ROT = (13, 15, 26, 6, 17, 29, 16, 24)
_SCHED = ((1, 2, 1), (2, 0, 2), (0, 1, 3), (1, 2, 4), (2, 0, 5))


def _np_rotl(x, d):
    return ((x << np.uint32(d)) | (x >> np.uint32(32 - d))).astype(np.uint32)


def _np_threefry2x32(k1, k2, x0, x1):
    ks = (np.uint32(k1), np.uint32(k2),
          np.uint32(np.uint32(k1) ^ np.uint32(k2) ^ np.uint32(0x1BD11BDA)))
    x0 = (x0 + ks[0]).astype(np.uint32)
    x1 = (x1 + ks[1]).astype(np.uint32)
    for g in range(5):
        for r in (_ROT[0:4] if g % 2 == 0 else _ROT[4:8]):
            x0 = (x0 + x1).astype(np.uint32)
            x1 = _np_rotl(x1, r)
            x1 = (x1 ^ x0).astype(np.uint32)
        a, b, c = _SCHED[g]
        x0 = (x0 + ks[a]).astype(np.uint32)
        x1 = (x1 + ks[b] + np.uint32(c)).astype(np.uint32)
    return x0, x1


def _np_subkeys(tw):
    cur = np.array([0, 42], np.uint32)
    out = np.empty((tw, 2), np.uint32)
    for t in range(tw):
        b1, b2 = _np_threefry2x32(cur[0], cur[1],
                                  np.array([0, 0], np.uint32),
                                  np.array([0, 1], np.uint32))
        cur = np.array([b1[0], b2[0]], np.uint32)
        out[t] = (b1[1], b2[1])
    return out


_SUBKEYS = np.concatenate([_np_subkeys(TW), np.zeros((1, 2), np.uint32)])


def _dot_tn(a, b):
    return jax.lax.dot_general(a, b, (((1,), (0,)), ((), ())),
                               preferred_element_type=jnp.float32)


def _snn_kernel(keys_ref, tw_ref, inp_ref, w1_ref, w2_ref, out_ref,
                p_ref, v_ref, h1_ref, h2_ref, ss_ref):
    B, F = inp_ref.shape

    p_ref[...] = (jax.lax.broadcasted_iota(jnp.uint32, (F, B), 1) * jnp.uint32(F)
                  + jax.lax.broadcasted_iota(jnp.uint32, (F, B), 0))
    v_ref[...] = jnp.ceil(inp_ref[...].T * jnp.float32(8388608.0)).astype(jnp.int32)
    h1_ref[...] = jnp.zeros_like(h1_ref)
    h2_ref[...] = jnp.zeros_like(h2_ref)
    ss_ref[...] = jnp.zeros_like(ss_ref)

    def step(i, carry):
        k1 = keys_ref[i, 0]
        k2 = keys_ref[i, 1]
        k3 = k1 ^ k2 ^ jnp.uint32(0x1BD11BDA)
        ks = (k1, k2, k3)

        x1 = p_ref[...] + k2
        x0 = x1 + k1
        x1 = ((x1 << jnp.uint32(13)) | (x1 >> jnp.uint32(19))) ^ x0
        for g in range(5):
            rots = _ROT[0:4] if g % 2 == 0 else _ROT[4:8]
            for r in (rots[1:] if g == 0 else rots):
                x0 = x0 + x1
                x1 = ((x1 << jnp.uint32(r)) | (x1 >> jnp.uint32(32 - r))) ^ x0
            a, b, c = _SCHED[g]
            x0 = x0 + ks[a]
            x1 = x1 + (ks[b] + jnp.uint32(c))
        bits = x0 ^ x1

        m = (bits >> jnp.uint32(9)).astype(jnp.int32)
        xT = jnp.where(v_ref[...] > m, jnp.float32(1.0), jnp.float32(0.0))

        h1 = h1_ref[...] + _dot_tn(w1_ref[...], xT)
        s1 = (h1 > THRESH).astype(jnp.float32)
        h1_ref[...] = jnp.where(h1 > THRESH, 0.0, h1) * DECAY

        h2 = h2_ref[...] + _dot_tn(w2_ref[...], s1)
        s2 = (h2 > THRESH).astype(jnp.float32)
        h2_ref[...] = jnp.where(h2 > THRESH, 0.0, h2) * DECAY

        ss_ref[...] = ss_ref[...] + s2
        return carry

    jax.lax.fori_loop(0, TW, step, 0, unroll=5)
    inv = jnp.float32(1.0) / tw_ref[0].astype(jnp.float32)
    out_ref[...] = ss_ref[...].T * inv


def kernel(input, W1, W2, time_window):
    B, F = input.shape
    H = W1.shape[0]
    O = W2.shape[0]

    tw_arr = jnp.reshape(jnp.asarray(time_window, dtype=jnp.int32), (1,))
    return pl.pallas_call(
        _snn_kernel,
        in_specs=[
            pl.BlockSpec(memory_space=pltpu.SMEM),
            pl.BlockSpec(memory_space=pltpu.SMEM),
            pl.BlockSpec(memory_space=pltpu.VMEM),
            pl.BlockSpec(memory_space=pltpu.VMEM),
            pl.BlockSpec(memory_space=pltpu.VMEM),
        ],
        out_specs=pl.BlockSpec(memory_space=pltpu.VMEM),
        out_shape=jax.ShapeDtypeStruct((B, O), jnp.float32),
        scratch_shapes=[
            pltpu.VMEM((F, B), jnp.uint32),
            pltpu.VMEM((F, B), jnp.int32),
            pltpu.VMEM((H, B), jnp.float32),
            pltpu.VMEM((O, B), jnp.float32),
            pltpu.VMEM((O, B), jnp.float32),
        ],
    )(jnp.asarray(_SUBKEYS), tw_arr, input, W1, W2)


# speedup vs baseline: 1.0065x; 1.0065x over previous
import numpy as np
import jax
import jax.numpy as jnp
from jax.experimental import pallas as pl
from jax.experimental.pallas import tpu as pltpu

THRESH = 0.5
DECAY = 0.2
TW = 20

_ROT = (13, 15, 26, 6, 17, 29, 16, 24)
_SCHED = ((1, 2, 1), (2, 0, 2), (0, 1, 3), (1, 2, 4), (2, 0, 5))


def _np_rotl(x, d):
    return ((x << np.uint32(d)) | (x >> np.uint32(32 - d))).astype(np.uint32)


def _np_threefry2x32(k1, k2, x0, x1):
    ks = (np.uint32(k1), np.uint32(k2),
          np.uint32(np.uint32(k1) ^ np.uint32(k2) ^ np.uint32(0x1BD11BDA)))
    x0 = (x0 + ks[0]).astype(np.uint32)
    x1 = (x1 + ks[1]).astype(np.uint32)
    for g in range(5):
        for r in (_ROT[0:4] if g % 2 == 0 else _ROT[4:8]):
            x0 = (x0 + x1).astype(np.uint32)
            x1 = _np_rotl(x1, r)
            x1 = (x1 ^ x0).astype(np.uint32)
        a, b, c = _SCHED[g]
        x0 = (x0 + ks[a]).astype(np.uint32)
        x1 = (x1 + ks[b] + np.uint32(c)).astype(np.uint32)
    return x0, x1


def _np_subkeys(tw):
    cur = np.array([0, 42], np.uint32)
    out = np.empty((tw, 2), np.uint32)
    for t in range(tw):
        b1, b2 = _np_threefry2x32(cur[0], cur[1],
                                  np.array([0, 0], np.uint32),
                                  np.array([0, 1], np.uint32))
        cur = np.array([b1[0], b2[0]], np.uint32)
        out[t] = (b1[1], b2[1])
    return out


_SUBKEYS = np.concatenate([_np_subkeys(TW), np.zeros((1, 2), np.uint32)])


def _dot_tn(a, b):
    return jax.lax.dot_general(a, b, (((1,), (0,)), ((), ())),
                               preferred_element_type=jnp.float32)


def _snn_kernel(keys_ref, tw_ref, inp_ref, w1_ref, w2_ref, out_ref,
                p_ref, v_ref, h1_ref, h2_ref, ss_ref):
    B, F = inp_ref.shape

    p_ref[...] = (jax.lax.broadcasted_iota(jnp.uint32, (F, B), 1) * jnp.uint32(F)
                  + jax.lax.broadcasted_iota(jnp.uint32, (F, B), 0))
    v_ref[...] = jnp.ceil(inp_ref[...].T * jnp.float32(8388608.0)).astype(jnp.int32)
    h1_ref[...] = jnp.zeros_like(h1_ref)
    h2_ref[...] = jnp.zeros_like(h2_ref)
    ss_ref[...] = jnp.zeros_like(ss_ref)

    def step(i, carry):
        k1 = keys_ref[i, 0]
        k2 = keys_ref[i, 1]
        k3 = k1 ^ k2 ^ jnp.uint32(0x1BD11BDA)
        ks = (k1, k2, k3)

        x1 = p_ref[...] + k2
        x0 = x1 + k1
        x1 = ((x1 << jnp.uint32(13)) | (x1 >> jnp.uint32(19))) ^ x0
        for g in range(5):
            rots = _ROT[0:4] if g % 2 == 0 else _ROT[4:8]
            for r in (rots[1:] if g == 0 else rots):
                x0 = x0 + x1
                x1 = ((x1 << jnp.uint32(r)) | (x1 >> jnp.uint32(32 - r))) ^ x0
            a, b, c = _SCHED[g]
            x0 = x0 + ks[a]
            x1 = x1 + (ks[b] + jnp.uint32(c))
        bits = x0 ^ x1

        m = (bits >> jnp.uint32(9)).astype(jnp.int32)
        xT = jnp.where(v_ref[...] > m, jnp.float32(1.0), jnp.float32(0.0))

        h1 = h1_ref[...] + _dot_tn(w1_ref[...], xT)
        s1 = (h1 > THRESH).astype(jnp.float32)
        h1_ref[...] = jnp.where(h1 > THRESH, 0.0, h1) * DECAY

        h2 = h2_ref[...] + _dot_tn(w2_ref[...], s1)
        s2 = (h2 > THRESH).astype(jnp.float32)
        h2_ref[...] = jnp.where(h2 > THRESH, 0.0, h2) * DECAY

        ss_ref[...] = ss_ref[...] + s2
        return carry

    jax.lax.fori_loop(0, TW, step, 0, unroll=10)
    inv = jnp.float32(1.0) / tw_ref[0].astype(jnp.float32)
    out_ref[...] = ss_ref[...].T * inv


def kernel(input, W1, W2, time_window):
    B, F = input.shape
    H = W1.shape[0]
    O = W2.shape[0]

    tw_arr = jnp.reshape(jnp.asarray(time_window, dtype=jnp.int32), (1,))
    return pl.pallas_call(
        _snn_kernel,
        in_specs=[
            pl.BlockSpec(memory_space=pltpu.SMEM),
            pl.BlockSpec(memory_space=pltpu.SMEM),
            pl.BlockSpec(memory_space=pltpu.VMEM),
            pl.BlockSpec(memory_space=pltpu.VMEM),
            pl.BlockSpec(memory_space=pltpu.VMEM),
        ],
        out_specs=pl.BlockSpec(memory_space=pltpu.VMEM),
        out_shape=jax.ShapeDtypeStruct((B, O), jnp.float32),
        scratch_shapes=[
            pltpu.VMEM((F, B), jnp.uint32),
            pltpu.VMEM((F, B), jnp.int32),
            pltpu.VMEM((H, B), jnp.float32),
            pltpu.VMEM((O, B), jnp.float32),
            pltpu.VMEM((O, B), jnp.float32),
        ],
    )(jnp.asarray(_SUBKEYS), tw_arr, input, W1, W2)


# speedup vs baseline: 1.0288x; 1.0221x over previous
import numpy as np
import jax
import jax.numpy as jnp
from jax.experimental import pallas as pl
from jax.experimental.pallas import tpu as pltpu

THRESH = 0.5
DECAY = 0.2
TW = 20

_ROT = (13, 15, 26, 6, 17, 29, 16, 24)
_SCHED = ((1, 2, 1), (2, 0, 2), (0, 1, 3), (1, 2, 4), (2, 0, 5))


def _np_rotl(x, d):
    return ((x << np.uint32(d)) | (x >> np.uint32(32 - d))).astype(np.uint32)


def _np_threefry2x32(k1, k2, x0, x1):
    ks = (np.uint32(k1), np.uint32(k2),
          np.uint32(np.uint32(k1) ^ np.uint32(k2) ^ np.uint32(0x1BD11BDA)))
    x0 = (x0 + ks[0]).astype(np.uint32)
    x1 = (x1 + ks[1]).astype(np.uint32)
    for g in range(5):
        for r in (_ROT[0:4] if g % 2 == 0 else _ROT[4:8]):
            x0 = (x0 + x1).astype(np.uint32)
            x1 = _np_rotl(x1, r)
            x1 = (x1 ^ x0).astype(np.uint32)
        a, b, c = _SCHED[g]
        x0 = (x0 + ks[a]).astype(np.uint32)
        x1 = (x1 + ks[b] + np.uint32(c)).astype(np.uint32)
    return x0, x1


def _np_subkeys(tw):
    cur = np.array([0, 42], np.uint32)
    out = np.empty((tw, 2), np.uint32)
    for t in range(tw):
        b1, b2 = _np_threefry2x32(cur[0], cur[1],
                                  np.array([0, 0], np.uint32),
                                  np.array([0, 1], np.uint32))
        cur = np.array([b1[0], b2[0]], np.uint32)
        out[t] = (b1[1], b2[1])
    return out


_SUBKEYS = np.concatenate([_np_subkeys(TW), np.zeros((1, 2), np.uint32)])


def _dot_tn(a, b):
    return jax.lax.dot_general(a, b, (((1,), (0,)), ((), ())),
                               preferred_element_type=jnp.float32)


def _snn_kernel(keys_ref, tw_ref, inp_ref, w1_ref, w2_ref, out_ref,
                p_ref, v_ref, h1_ref, h2_ref, ss_ref):
    B, F = inp_ref.shape

    p_ref[...] = (jax.lax.broadcasted_iota(jnp.uint32, (F, B), 1) * jnp.uint32(F)
                  + jax.lax.broadcasted_iota(jnp.uint32, (F, B), 0))
    v_ref[...] = jnp.ceil(inp_ref[...].T * jnp.float32(8388608.0)).astype(jnp.int32)
    h1_ref[...] = jnp.zeros_like(h1_ref)
    h2_ref[...] = jnp.zeros_like(h2_ref)
    ss_ref[...] = jnp.zeros_like(ss_ref)

    def step(i, carry):
        k1 = keys_ref[i, 0]
        k2 = keys_ref[i, 1]
        k3 = k1 ^ k2 ^ jnp.uint32(0x1BD11BDA)
        ks = (k1, k2, k3)

        x1 = p_ref[...] + k2
        x0 = x1 + k1
        x1 = ((x1 << jnp.uint32(13)) | (x1 >> jnp.uint32(19))) ^ x0
        for g in range(5):
            rots = _ROT[0:4] if g % 2 == 0 else _ROT[4:8]
            for r in (rots[1:] if g == 0 else rots):
                x0 = x0 + x1
                x1 = ((x1 << jnp.uint32(r)) | (x1 >> jnp.uint32(32 - r))) ^ x0
            a, b, c = _SCHED[g]
            x0 = x0 + ks[a]
            x1 = x1 + (ks[b] + jnp.uint32(c))
        bits = x0 ^ x1

        m = (bits >> jnp.uint32(9)).astype(jnp.int32)
        xT = jnp.where(v_ref[...] > m, jnp.float32(1.0), jnp.float32(0.0))

        h1 = h1_ref[...] + _dot_tn(w1_ref[...], xT)
        s1 = (h1 > THRESH).astype(jnp.float32)
        h1_ref[...] = jnp.where(h1 > THRESH, 0.0, h1) * DECAY

        h2 = h2_ref[...] + _dot_tn(w2_ref[...], s1)
        s2 = (h2 > THRESH).astype(jnp.float32)
        h2_ref[...] = jnp.where(h2 > THRESH, 0.0, h2) * DECAY

        ss_ref[...] = ss_ref[...] + s2
        return carry

    jax.lax.fori_loop(0, TW, step, 0, unroll=TW)
    inv = jnp.float32(1.0) / tw_ref[0].astype(jnp.float32)
    out_ref[...] = ss_ref[...].T * inv


def kernel(input, W1, W2, time_window):
    B, F = input.shape
    H = W1.shape[0]
    O = W2.shape[0]

    tw_arr = jnp.reshape(jnp.asarray(time_window, dtype=jnp.int32), (1,))
    return pl.pallas_call(
        _snn_kernel,
        in_specs=[
            pl.BlockSpec(memory_space=pltpu.SMEM),
            pl.BlockSpec(memory_space=pltpu.SMEM),
            pl.BlockSpec(memory_space=pltpu.VMEM),
            pl.BlockSpec(memory_space=pltpu.VMEM),
            pl.BlockSpec(memory_space=pltpu.VMEM),
        ],
        out_specs=pl.BlockSpec(memory_space=pltpu.VMEM),
        out_shape=jax.ShapeDtypeStruct((B, O), jnp.float32),
        scratch_shapes=[
            pltpu.VMEM((F, B), jnp.uint32),
            pltpu.VMEM((F, B), jnp.int32),
            pltpu.VMEM((H, B), jnp.float32),
            pltpu.VMEM((O, B), jnp.float32),
            pltpu.VMEM((O, B), jnp.float32),
        ],
    )(jnp.asarray(_SUBKEYS), tw_arr, input, W1, W2)
